# baseline (speedup 1.0000x reference)
import jax
import jax.numpy as jnp
from jax import lax
from jax.experimental import pallas as pl
from jax.experimental.pallas import tpu as pltpu
from jax.experimental.pallas import tpu_sc as plsc

VOCAB = 100000
D_MODEL = 512
BATCH = 1024
SEQ = 50
SEQP = 56

_NC = 2
_NS = 16
_NW = _NC * _NS
_ROWS = BATCH * SEQP
_RPW = _ROWS // _NW
_CH = 32
_NCHUNK = _RPW // _CH
_LANES = 16
_DCHUNKS = D_MODEL // _LANES
_NBUF = 4
_NITER = _NCHUNK // _NBUF


def _pe_table():
    pos = jnp.arange(SEQ, dtype=jnp.float32)[:, None]
    hid = jnp.arange(D_MODEL, dtype=jnp.float32)[None, :]
    angle = pos / jnp.power(10000.0, 2.0 * jnp.floor(hid / 2.0) / D_MODEL)
    even = (jnp.arange(SEQ) % 2 == 0)[:, None]
    pe = jnp.where(even, jnp.sin(angle), jnp.cos(angle))
    return jnp.pad(pe, ((0, SEQP - SEQ), (0, 0)))


def _body(x_hbm, w_hbm, pe_hbm, out_hbm,
          pe_v, idx_v, b0, b1, b2, b3, mask_v,
          g0, g1, g2, g3, s0, s1, s2, s3):
    wid = lax.axis_index("s") * _NC + lax.axis_index("c")
    row0 = wid * _RPW

    pltpu.sync_copy(pe_hbm, pe_v)
    pltpu.sync_copy(x_hbm.at[pl.ds(row0, _RPW)], idx_v)

    bufs = (b0, b1, b2, b3)
    gsem = (g0, g1, g2, g3)
    ssem = (s0, s1, s2, s3)

    def gather_desc(t, j):
        return pltpu.make_async_copy(
            w_hbm.at[idx_v.at[pl.ds(t * _CH, _CH)]], bufs[j], gsem[j]
        )

    def scatter_desc(t, j):
        return pltpu.make_async_copy(
            bufs[j], out_hbm.at[pl.ds(row0 + t * _CH, _CH), :], ssem[j]
        )

    def compute(t, j):
        buf = bufs[j]
        for k in range(_CH // _LANES):
            ksl = pl.ds(k * _LANES, _LANES)
            mask_v[ksl] = jnp.minimum(
                idx_v[pl.ds(t * _CH + k * _LANES, _LANES)], 1
            ).astype(jnp.float32)

        p0 = lax.rem(row0 + t * _CH, SEQP)

        def per_row(r, p):
            splat = jnp.full((_LANES,), r, dtype=jnp.int32)
            mv = plsc.load_gather(mask_v, [splat])
            for d in range(_DCHUNKS):
                sl = pl.ds(d * _LANES, _LANES)
                buf[r, sl] = buf[r, sl] * mv + pe_v[p, sl]
            pn = p + 1
            return jnp.where(pn == SEQP, 0, pn)

        lax.fori_loop(0, _CH, per_row, p0, unroll=1)

    gather_desc(0, 0).start()
    gather_desc(1, 1).start()

    def outer(i, carry):
        c0 = i * _NBUF
        for j in range(_NBUF):
            t = c0 + j
            jn = (j + 2) % _NBUF

            @pl.when(t >= 2)
            def _ws(t=t, jn=jn):
                scatter_desc(t - 2, jn).wait()

            @pl.when(t + 2 < _NCHUNK)
            def _gs(t=t, jn=jn):
                gather_desc(t + 2, jn).start()

            gather_desc(t, j).wait()
            compute(t, j)
            scatter_desc(t, j).start()
        return carry

    lax.fori_loop(0, _NITER, outer, 0, unroll=1)
    scatter_desc(_NCHUNK - 2, (_NCHUNK - 2) % _NBUF).wait()
    scatter_desc(_NCHUNK - 1, (_NCHUNK - 1) % _NBUF).wait()


@jax.jit
def _embed(xr, w, pe):
    mesh = plsc.VectorSubcoreMesh(core_axis_name="c", subcore_axis_name="s")
    f = pl.kernel(
        _body,
        out_type=jax.ShapeDtypeStruct((_ROWS, D_MODEL), jnp.float32),
        mesh=mesh,
        compiler_params=pltpu.CompilerParams(needs_layout_passes=False),
        scratch_types=[
            pltpu.VMEM((SEQP, D_MODEL), jnp.float32),
            pltpu.VMEM((_RPW,), jnp.int32),
            pltpu.VMEM((_CH, D_MODEL), jnp.float32),
            pltpu.VMEM((_CH, D_MODEL), jnp.float32),
            pltpu.VMEM((_CH, D_MODEL), jnp.float32),
            pltpu.VMEM((_CH, D_MODEL), jnp.float32),
            pltpu.VMEM((_CH,), jnp.float32),
            pltpu.SemaphoreType.DMA,
            pltpu.SemaphoreType.DMA,
            pltpu.SemaphoreType.DMA,
            pltpu.SemaphoreType.DMA,
            pltpu.SemaphoreType.DMA,
            pltpu.SemaphoreType.DMA,
            pltpu.SemaphoreType.DMA,
            pltpu.SemaphoreType.DMA,
        ],
    )
    return f(xr, w, pe)


def kernel(x, weight):
    pe = _pe_table()
    xp = jnp.pad(
        x.astype(jnp.int32), ((0, 0), (0, SEQP - SEQ)), constant_values=1
    ).reshape(_ROWS)
    out = _embed(xp, weight, pe).reshape(BATCH, SEQP, D_MODEL)
    return out[:, :SEQ, :]


# speedup vs baseline: 1.2384x; 1.2384x over previous
import jax
import jax.numpy as jnp
from jax import lax
from jax.experimental import pallas as pl
from jax.experimental.pallas import tpu as pltpu
from jax.experimental.pallas import tpu_sc as plsc

VOCAB = 100000
D_MODEL = 512
BATCH = 1024
SEQ = 50

_NC = 2
_NS = 16
_NW = _NC * _NS
_ROWS = BATCH * SEQ
_RPW = _ROWS // _NW
_CH = 32
_NCHUNK = _RPW // _CH
_LANES = 16
_DCHUNKS = D_MODEL // _LANES
_NBUF = 5
_NITER = _NCHUNK // _NBUF


def _pe_table():
    pos = jnp.arange(SEQ, dtype=jnp.float32)[:, None]
    hid = jnp.arange(D_MODEL, dtype=jnp.float32)[None, :]
    angle = pos / jnp.power(10000.0, 2.0 * jnp.floor(hid / 2.0) / D_MODEL)
    even = (jnp.arange(SEQ) % 2 == 0)[:, None]
    return jnp.where(even, jnp.sin(angle), jnp.cos(angle))


def _body(x_hbm, w_hbm, pe_hbm, out_hbm,
          pe_v, idx_v, b0, b1, b2, b3, b4, mask_v,
          g0, g1, g2, g3, g4, s0, s1, s2, s3, s4):
    wid = lax.axis_index("s") * _NC + lax.axis_index("c")
    row0 = wid * _RPW

    pltpu.sync_copy(pe_hbm, pe_v)
    pltpu.sync_copy(x_hbm.at[pl.ds(row0, _RPW)], idx_v)

    bufs = (b0, b1, b2, b3, b4)
    gsem = (g0, g1, g2, g3, g4)
    ssem = (s0, s1, s2, s3, s4)

    def gather_desc(t, j):
        return pltpu.make_async_copy(
            w_hbm.at[idx_v.at[pl.ds(t * _CH, _CH)]], bufs[j], gsem[j]
        )

    def scatter_desc(t, j):
        return pltpu.make_async_copy(
            bufs[j], out_hbm.at[pl.ds(row0 + t * _CH, _CH), :], ssem[j]
        )

    def compute(t, j):
        buf = bufs[j]
        for k in range(_CH // _LANES):
            ksl = pl.ds(k * _LANES, _LANES)
            mask_v[ksl] = jnp.minimum(
                idx_v[pl.ds(t * _CH + k * _LANES, _LANES)], 1
            ).astype(jnp.float32)

        p0 = lax.rem(row0 + t * _CH, SEQ)

        def per_row(r, p):
            splat = jnp.full((_LANES,), r, dtype=jnp.int32)
            mv = plsc.load_gather(mask_v, [splat])
            for d in range(_DCHUNKS):
                sl = pl.ds(d * _LANES, _LANES)
                buf[r, sl] = buf[r, sl] * mv + pe_v[p, sl]
            pn = p + 1
            return jnp.where(pn == SEQ, 0, pn)

        lax.fori_loop(0, _CH, per_row, p0, unroll=1)

    gather_desc(0, 0).start()
    gather_desc(1, 1).start()

    def outer(i, carry):
        c0 = i * _NBUF
        for j in range(_NBUF):
            t = c0 + j
            jn = (j + 2) % _NBUF

            @pl.when(t >= 3)
            def _ws(t=t, jn=jn):
                scatter_desc(t - 3, jn).wait()

            @pl.when(t + 2 < _NCHUNK)
            def _gs(t=t, jn=jn):
                gather_desc(t + 2, jn).start()

            gather_desc(t, j).wait()
            compute(t, j)
            scatter_desc(t, j).start()
        return carry

    lax.fori_loop(0, _NITER, outer, 0, unroll=1)
    scatter_desc(_NCHUNK - 3, (_NCHUNK - 3) % _NBUF).wait()
    scatter_desc(_NCHUNK - 2, (_NCHUNK - 2) % _NBUF).wait()
    scatter_desc(_NCHUNK - 1, (_NCHUNK - 1) % _NBUF).wait()


@jax.jit
def _embed(xr, w, pe):
    mesh = plsc.VectorSubcoreMesh(core_axis_name="c", subcore_axis_name="s")
    f = pl.kernel(
        _body,
        out_type=jax.ShapeDtypeStruct((_ROWS, D_MODEL), jnp.float32),
        mesh=mesh,
        compiler_params=pltpu.CompilerParams(needs_layout_passes=False),
        scratch_types=[
            pltpu.VMEM((SEQ, D_MODEL), jnp.float32),
            pltpu.VMEM((_RPW,), jnp.int32),
            pltpu.VMEM((_CH, D_MODEL), jnp.float32),
            pltpu.VMEM((_CH, D_MODEL), jnp.float32),
            pltpu.VMEM((_CH, D_MODEL), jnp.float32),
            pltpu.VMEM((_CH, D_MODEL), jnp.float32),
            pltpu.VMEM((_CH, D_MODEL), jnp.float32),
            pltpu.VMEM((_CH,), jnp.float32),
            pltpu.SemaphoreType.DMA,
            pltpu.SemaphoreType.DMA,
            pltpu.SemaphoreType.DMA,
            pltpu.SemaphoreType.DMA,
            pltpu.SemaphoreType.DMA,
            pltpu.SemaphoreType.DMA,
            pltpu.SemaphoreType.DMA,
            pltpu.SemaphoreType.DMA,
            pltpu.SemaphoreType.DMA,
            pltpu.SemaphoreType.DMA,
        ],
    )
    return f(xr, w, pe)


def kernel(x, weight):
    pe = _pe_table()
    xr = x.astype(jnp.int32).reshape(_ROWS)
    return _embed(xr, weight, pe).reshape(BATCH, SEQ, D_MODEL)
